# baseline (speedup 1.0000x reference)
import functools

import jax
import jax.numpy as jnp
from jax import lax
from jax.experimental import pallas as pl
from jax.experimental.pallas import tpu as pltpu
from jax.experimental.pallas import tpu_sc as plsc

HIDDEN = 128
N_HW = 169
N_HW_PAD = 176
N_BKT = 24
N_ROWS_PAD = N_HW_PAD * N_BKT
_GRID = 16


def _tc_body(ts_ref, t_ref, mk_ref,
             h_emb, w_emb, i_emb, t_emb,
             nh, nw, ni, nt, ph, pw, pi, ptt,
             idx_ref, tab_ref):
    f32 = jnp.float32
    hb = tab_ref.shape[0]
    pid = pl.program_id(0)
    i = pid * hb + lax.broadcasted_iota(jnp.int32, (hb, 1), 0)
    valid = i >= 1
    hr = jnp.where(valid, (i - 1) // 7 + 1, 0)
    wd = jnp.where(valid, (i - 1) % 7 + 1, 0)
    iw = jnp.where(valid, jnp.where(wd >= 6, 2, 1), 0)

    oh_h = (hr == lax.broadcasted_iota(jnp.int32, (hb, 25), 1)).astype(f32)
    oh_w = (wd == lax.broadcasted_iota(jnp.int32, (hb, 8), 1)).astype(f32)
    oh_i = (iw == lax.broadcasted_iota(jnp.int32, (hb, 3), 1)).astype(f32)

    Hp = jnp.dot(h_emb[...] * nh[...], ph[...], preferred_element_type=f32)
    Wp = jnp.dot(w_emb[...] * nw[...], pw[...], preferred_element_type=f32)
    Ip = jnp.dot(i_emb[...] * ni[...], pi[...], preferred_element_type=f32)
    Tp = jnp.dot(t_emb[...] * nt[...], ptt[...], preferred_element_type=f32)

    A = (jnp.dot(oh_w, Wp, preferred_element_type=f32)
         + jnp.dot(oh_h, Hp, preferred_element_type=f32)
         + jnp.dot(oh_i, Ip, preferred_element_type=f32))

    ss_h = jnp.sum(h_emb[...] * h_emb[...], axis=1, keepdims=True)
    ss_w = jnp.sum(w_emb[...] * w_emb[...], axis=1, keepdims=True)
    ss_i = jnp.sum(i_emb[...] * i_emb[...], axis=1, keepdims=True)
    ssA = (jnp.dot(oh_w, ss_w, preferred_element_type=f32)
           + jnp.dot(oh_h, ss_h, preferred_element_type=f32)
           + jnp.dot(oh_i, ss_i, preferred_element_type=f32))
    ssC = jnp.sum(t_emb[...] * t_emb[...], axis=1, keepdims=True)

    eps = jnp.float32(jnp.finfo(jnp.float32).eps)
    mean_sq = (ssA.reshape(hb, 1, 1) + ssC.reshape(1, N_BKT, 1)) * jnp.float32(1.0 / 34.0)
    rms = lax.rsqrt(mean_sq + eps)
    tab_ref[...] = rms * (A.reshape(hb, 1, HIDDEN) + Tp.reshape(1, N_BKT, HIDDEN))

    ts = ts_ref[...]
    m = ts != 0
    hour = ts // 3600 % 24 + 1
    weekday = (ts // 86400 + 4) % 7 + 1
    hw = jnp.where(m, (hour - 1) * 7 + weekday, 0)
    bf = jnp.floor(jnp.log1p(t_ref[...].astype(f32))).astype(jnp.int32)
    bucket = jnp.clip(bf, 0, 21) + 1
    bucket = jnp.where(mk_ref[...] != 0, bucket, 0)
    idx_ref[...] = hw * N_BKT + bucket


def _tc_prologue(timestamps, t_from_prev, mask_i32,
                 h_emb, w_emb, i_emb, t_emb, norm_w, proj_w):
    B, L = timestamps.shape
    projT = proj_w.T
    nh, nw, ni, nt = (norm_w[8:16].reshape(1, 8), norm_w[0:8].reshape(1, 8),
                      norm_w[16:18].reshape(1, 2), norm_w[18:34].reshape(1, 16))
    ph, pw, pi, ptt = projT[8:16], projT[0:8], projT[16:18], projT[18:34]
    t_emb = t_emb[:N_BKT]
    hb = N_HW_PAD // _GRID
    bb = B // _GRID
    dspec = pl.BlockSpec((bb, L), lambda p: (p, 0))
    full = lambda s: pl.BlockSpec(s, lambda p: (0,) * len(s))
    idx, table = pl.pallas_call(
        _tc_body,
        grid=(_GRID,),
        in_specs=[
            dspec, dspec, dspec,
            full((25, 8)), full((8, 8)), full((3, 2)), full((N_BKT, 16)),
            full((1, 8)), full((1, 8)), full((1, 2)), full((1, 16)),
            full((8, HIDDEN)), full((8, HIDDEN)), full((2, HIDDEN)), full((16, HIDDEN)),
        ],
        out_specs=[dspec, pl.BlockSpec((hb, N_BKT, HIDDEN), lambda p: (p, 0, 0))],
        out_shape=[jax.ShapeDtypeStruct((B, L), jnp.int32),
                   jax.ShapeDtypeStruct((N_HW_PAD, N_BKT, HIDDEN), jnp.float32)],
    )(timestamps, t_from_prev, mask_i32,
      h_emb, w_emb, i_emb, t_emb, nh, nw, ni, nt, ph, pw, pi, ptt)
    return idx.reshape(B * L), table.reshape(N_ROWS_PAD, HIDDEN)


_CH = 128
_NBUF = 4


def _make_gather(n_rows):
    info = plsc.get_sparse_core_info()
    nsub = info.num_subcores
    nworkers = info.num_cores * nsub
    per_w = n_rows // nworkers
    steps = per_w // _CH
    stage_rows = N_ROWS_PAD // nsub
    assert steps % _NBUF == 0 and steps >= 2 * _NBUF
    mesh = plsc.VectorSubcoreMesh(core_axis_name="c", subcore_axis_name="s")

    @functools.partial(
        pl.kernel,
        mesh=mesh,
        out_type=jax.ShapeDtypeStruct((n_rows, HIDDEN), jnp.float32),
        scratch_types=(
            [pltpu.VMEM((per_w,), jnp.int32),
             pltpu.VMEM((_NBUF, _CH, HIDDEN), jnp.float32),
             pltpu.VMEM_SHARED((N_ROWS_PAD, HIDDEN), jnp.float32)]
            + [pltpu.SemaphoreType.DMA] * (2 * _NBUF)
        ),
    )
    def gather(table_hbm, idx_hbm, out_hbm, idx_v, rows_v, table_sp, *sems):
        sg, sw = sems[:_NBUF], sems[_NBUF:]
        sid = lax.axis_index("s")
        wid = sid * info.num_cores + lax.axis_index("c")
        base = wid * per_w

        srow = pl.multiple_of(sid * stage_rows, stage_rows)
        pltpu.sync_copy(table_hbm.at[pl.ds(srow, stage_rows)],
                        table_sp.at[pl.ds(srow, stage_rows)])
        pltpu.sync_copy(idx_hbm.at[pl.ds(base, per_w)], idx_v)
        plsc.subcore_barrier()

        def start_gather(g, b, sem):
            off = pl.multiple_of(g * _CH, _CH)
            pltpu.async_copy(table_sp.at[idx_v.at[pl.ds(off, _CH)]],
                             rows_v.at[b], sem)

        def start_write(g, b, sem):
            off = pl.multiple_of(base + g * _CH, _CH)
            pltpu.async_copy(rows_v.at[b], out_hbm.at[pl.ds(off, _CH)], sem)

        def wait_chunk(sem):
            pltpu.make_async_copy(out_hbm.at[pl.ds(0, _CH)], rows_v.at[0], sem).wait()

        for b in range(_NBUF):
            start_gather(b, b, sg[b])
        wait_chunk(sg[0])
        start_write(0, 0, sw[0])

        def outer(og, carry):
            for b in range(_NBUF):
                g = og * _NBUF + b
                wait_chunk(sw[b])
                start_gather(g, b, sg[b])
                b2 = (b + 1) % _NBUF
                wait_chunk(sg[b2])
                start_write(g - (_NBUF - 1), b2, sw[b2])
            return carry

        lax.fori_loop(1, steps // _NBUF, outer, 0)

        for j in range(steps - (_NBUF - 1), steps):
            b = j % _NBUF
            wait_chunk(sg[b])
            start_write(j, b, sw[b])
        for b in range(_NBUF):
            wait_chunk(sw[b])

    return gather


def kernel(timestamps, t_from_prev, mask, hour_emb, weekday_emb, isweekend_emb,
           t_from_prev_emb, norm_w, proj_w):
    B, L = timestamps.shape
    idx, table = _tc_prologue(timestamps, t_from_prev, mask.astype(jnp.int32),
                              hour_emb, weekday_emb, isweekend_emb,
                              t_from_prev_emb, norm_w, proj_w)
    out = _make_gather(B * L)(table, idx)
    return out.reshape(B, L, HIDDEN)


# speedup vs baseline: 1.0047x; 1.0047x over previous
import functools

import jax
import jax.numpy as jnp
from jax import lax
from jax.experimental import pallas as pl
from jax.experimental.pallas import tpu as pltpu
from jax.experimental.pallas import tpu_sc as plsc

HIDDEN = 128
N_HW = 169
N_HW_PAD = 176
N_BKT = 24
N_ROWS_PAD = N_HW_PAD * N_BKT
_GRID = 16


def _table_body(h_emb, w_emb, i_emb, t_emb,
                nh, nw, ni, nt, ph, pw, pi, ptt, tab_ref):
    f32 = jnp.float32
    hb = tab_ref.shape[0]
    pid = pl.program_id(0)
    i = pid * hb + lax.broadcasted_iota(jnp.int32, (hb, 1), 0)
    valid = i >= 1
    hr = jnp.where(valid, (i - 1) // 7 + 1, 0)
    wd = jnp.where(valid, (i - 1) % 7 + 1, 0)
    iw = jnp.where(valid, jnp.where(wd >= 6, 2, 1), 0)

    oh_h = (hr == lax.broadcasted_iota(jnp.int32, (hb, 25), 1)).astype(f32)
    oh_w = (wd == lax.broadcasted_iota(jnp.int32, (hb, 8), 1)).astype(f32)
    oh_i = (iw == lax.broadcasted_iota(jnp.int32, (hb, 3), 1)).astype(f32)

    Hp = jnp.dot(h_emb[...] * nh[...], ph[...], preferred_element_type=f32)
    Wp = jnp.dot(w_emb[...] * nw[...], pw[...], preferred_element_type=f32)
    Ip = jnp.dot(i_emb[...] * ni[...], pi[...], preferred_element_type=f32)
    Tp = jnp.dot(t_emb[...] * nt[...], ptt[...], preferred_element_type=f32)

    A = (jnp.dot(oh_w, Wp, preferred_element_type=f32)
         + jnp.dot(oh_h, Hp, preferred_element_type=f32)
         + jnp.dot(oh_i, Ip, preferred_element_type=f32))

    ss_h = jnp.sum(h_emb[...] * h_emb[...], axis=1, keepdims=True)
    ss_w = jnp.sum(w_emb[...] * w_emb[...], axis=1, keepdims=True)
    ss_i = jnp.sum(i_emb[...] * i_emb[...], axis=1, keepdims=True)
    ssA = (jnp.dot(oh_w, ss_w, preferred_element_type=f32)
           + jnp.dot(oh_h, ss_h, preferred_element_type=f32)
           + jnp.dot(oh_i, ss_i, preferred_element_type=f32))
    ssC = jnp.sum(t_emb[...] * t_emb[...], axis=1, keepdims=True)

    eps = jnp.float32(jnp.finfo(jnp.float32).eps)
    mean_sq = (ssA.reshape(hb, 1, 1) + ssC.reshape(1, N_BKT, 1)) * jnp.float32(1.0 / 34.0)
    rms = lax.rsqrt(mean_sq + eps)
    tab_ref[...] = rms * (A.reshape(hb, 1, HIDDEN) + Tp.reshape(1, N_BKT, HIDDEN))


def _idx_body(ts_ref, t_ref, mk_ref, idx_ref):
    f32 = jnp.float32
    ts = ts_ref[...]
    m = ts != 0
    hour = ts // 3600 % 24 + 1
    weekday = (ts // 86400 + 4) % 7 + 1
    hw = jnp.where(m, (hour - 1) * 7 + weekday, 0)
    bf = jnp.floor(jnp.log1p(t_ref[...].astype(f32))).astype(jnp.int32)
    bucket = jnp.clip(bf, 0, 21) + 1
    bucket = jnp.where(mk_ref[...] != 0, bucket, 0)
    idx_ref[...] = hw * N_BKT + bucket


def _tc_prologue(timestamps, t_from_prev, mask_i32,
                 h_emb, w_emb, i_emb, t_emb, norm_w, proj_w):
    B, L = timestamps.shape
    projT = proj_w.T
    nh, nw, ni, nt = (norm_w[8:16].reshape(1, 8), norm_w[0:8].reshape(1, 8),
                      norm_w[16:18].reshape(1, 2), norm_w[18:34].reshape(1, 16))
    ph, pw, pi, ptt = projT[8:16], projT[0:8], projT[16:18], projT[18:34]
    t_emb = t_emb[:N_BKT]
    hb = N_HW_PAD // _GRID
    full = lambda s: pl.BlockSpec(s, lambda p: (0,) * len(s))
    table = pl.pallas_call(
        _table_body,
        grid=(_GRID,),
        in_specs=[
            full((25, 8)), full((8, 8)), full((3, 2)), full((N_BKT, 16)),
            full((1, 8)), full((1, 8)), full((1, 2)), full((1, 16)),
            full((8, HIDDEN)), full((8, HIDDEN)), full((2, HIDDEN)), full((16, HIDDEN)),
        ],
        out_specs=pl.BlockSpec((hb, N_BKT, HIDDEN), lambda p: (p, 0, 0)),
        out_shape=jax.ShapeDtypeStruct((N_HW_PAD, N_BKT, HIDDEN), jnp.float32),
    )(h_emb, w_emb, i_emb, t_emb, nh, nw, ni, nt, ph, pw, pi, ptt)
    bb = B // 8
    dspec = pl.BlockSpec((bb, L), lambda p: (p, 0))
    idx = pl.pallas_call(
        _idx_body,
        grid=(8,),
        in_specs=[dspec, dspec, dspec],
        out_specs=dspec,
        out_shape=jax.ShapeDtypeStruct((B, L), jnp.int32),
    )(timestamps, t_from_prev, mask_i32)
    return idx.reshape(B * L), table.reshape(N_ROWS_PAD, HIDDEN)


_CH = 128
_NBUF = 4


def _make_gather(n_rows):
    info = plsc.get_sparse_core_info()
    nsub = info.num_subcores
    nworkers = info.num_cores * nsub
    per_w = n_rows // nworkers
    steps = per_w // _CH
    stage_rows = N_ROWS_PAD // nsub
    assert steps % _NBUF == 0 and steps >= 2 * _NBUF
    mesh = plsc.VectorSubcoreMesh(core_axis_name="c", subcore_axis_name="s")

    @functools.partial(
        pl.kernel,
        mesh=mesh,
        out_type=jax.ShapeDtypeStruct((n_rows, HIDDEN), jnp.float32),
        scratch_types=(
            [pltpu.VMEM((per_w,), jnp.int32),
             pltpu.VMEM((_NBUF, _CH, HIDDEN), jnp.float32),
             pltpu.VMEM_SHARED((N_ROWS_PAD, HIDDEN), jnp.float32)]
            + [pltpu.SemaphoreType.DMA] * (2 * _NBUF)
        ),
    )
    def gather(table_hbm, idx_hbm, out_hbm, idx_v, rows_v, table_sp, *sems):
        sg, sw = sems[:_NBUF], sems[_NBUF:]
        sid = lax.axis_index("s")
        wid = sid * info.num_cores + lax.axis_index("c")
        base = wid * per_w

        srow = pl.multiple_of(sid * stage_rows, stage_rows)
        pltpu.sync_copy(table_hbm.at[pl.ds(srow, stage_rows)],
                        table_sp.at[pl.ds(srow, stage_rows)])
        pltpu.sync_copy(idx_hbm.at[pl.ds(base, per_w)], idx_v)
        plsc.subcore_barrier()

        def start_gather(g, b, sem):
            off = pl.multiple_of(g * _CH, _CH)
            pltpu.async_copy(table_sp.at[idx_v.at[pl.ds(off, _CH)]],
                             rows_v.at[b], sem)

        def start_write(g, b, sem):
            off = pl.multiple_of(base + g * _CH, _CH)
            pltpu.async_copy(rows_v.at[b], out_hbm.at[pl.ds(off, _CH)], sem)

        def wait_chunk(sem):
            pltpu.make_async_copy(out_hbm.at[pl.ds(0, _CH)], rows_v.at[0], sem).wait()

        for b in range(_NBUF):
            start_gather(b, b, sg[b])
        wait_chunk(sg[0])
        start_write(0, 0, sw[0])

        def outer(og, carry):
            for b in range(_NBUF):
                g = og * _NBUF + b
                wait_chunk(sw[b])
                start_gather(g, b, sg[b])
                b2 = (b + 1) % _NBUF
                wait_chunk(sg[b2])
                start_write(g - (_NBUF - 1), b2, sw[b2])
            return carry

        lax.fori_loop(1, steps // _NBUF, outer, 0)

        for j in range(steps - (_NBUF - 1), steps):
            b = j % _NBUF
            wait_chunk(sg[b])
            start_write(j, b, sw[b])
        for b in range(_NBUF):
            wait_chunk(sw[b])

    return gather


def kernel(timestamps, t_from_prev, mask, hour_emb, weekday_emb, isweekend_emb,
           t_from_prev_emb, norm_w, proj_w):
    B, L = timestamps.shape
    idx, table = _tc_prologue(timestamps, t_from_prev, mask.astype(jnp.int32),
                              hour_emb, weekday_emb, isweekend_emb,
                              t_from_prev_emb, norm_w, proj_w)
    out = _make_gather(B * L)(table, idx)
    return out.reshape(B, L, HIDDEN)


# speedup vs baseline: 1.1692x; 1.1638x over previous
import functools

import jax
import jax.numpy as jnp
from jax import lax
from jax.experimental import pallas as pl
from jax.experimental.pallas import tpu as pltpu
from jax.experimental.pallas import tpu_sc as plsc

HIDDEN = 128
N_HW = 169
N_BKT = 23
N_ROWS = N_HW * N_BKT
N_ROWS_PAD = 3968
_GRID = 13


def _dot_nt(a, b):
    return lax.dot_general(a, b, (((1,), (1,)), ((), ())),
                           preferred_element_type=jnp.float32)


def _table_body(h_emb, w_emb, i_emb, t_emb, norm2d, proj_w, tab_ref):
    f32 = jnp.float32
    hb = tab_ref.shape[0]
    pid = pl.program_id(0)
    i = pid * hb + lax.broadcasted_iota(jnp.int32, (hb, 1), 0)
    valid = i >= 1
    hr = jnp.where(valid, (i - 1) // 7 + 1, 0)
    wd = jnp.where(valid, (i - 1) % 7 + 1, 0)
    iw = jnp.where(valid, jnp.where(wd >= 6, 2, 1), 0)

    oh_h = (hr == lax.broadcasted_iota(jnp.int32, (hb, 25), 1)).astype(f32)
    oh_w = (wd == lax.broadcasted_iota(jnp.int32, (hb, 8), 1)).astype(f32)
    oh_i = (iw == lax.broadcasted_iota(jnp.int32, (hb, 3), 1)).astype(f32)

    nrm = norm2d[...]
    prj = proj_w[...]
    t_sl = t_emb[0:N_BKT, :]
    Hp = _dot_nt(h_emb[...] * nrm[:, 8:16], prj[:, 8:16])
    Wp = _dot_nt(w_emb[...] * nrm[:, 0:8], prj[:, 0:8])
    Ip = _dot_nt(i_emb[...] * nrm[:, 16:18], prj[:, 16:18])
    Tp = _dot_nt(t_sl * nrm[:, 18:34], prj[:, 18:34])

    A = (jnp.dot(oh_w, Wp, preferred_element_type=f32)
         + jnp.dot(oh_h, Hp, preferred_element_type=f32)
         + jnp.dot(oh_i, Ip, preferred_element_type=f32))

    ss_h = jnp.sum(h_emb[...] * h_emb[...], axis=1, keepdims=True)
    ss_w = jnp.sum(w_emb[...] * w_emb[...], axis=1, keepdims=True)
    ss_i = jnp.sum(i_emb[...] * i_emb[...], axis=1, keepdims=True)
    ssA = (jnp.dot(oh_w, ss_w, preferred_element_type=f32)
           + jnp.dot(oh_h, ss_h, preferred_element_type=f32)
           + jnp.dot(oh_i, ss_i, preferred_element_type=f32))
    ssC = jnp.sum(t_sl * t_sl, axis=1, keepdims=True)

    eps = jnp.float32(jnp.finfo(jnp.float32).eps)
    mean_sq = (ssA.reshape(hb, 1, 1) + ssC.reshape(1, N_BKT, 1)) * jnp.float32(1.0 / 34.0)
    rms = lax.rsqrt(mean_sq + eps)
    tab_ref[...] = rms * (A.reshape(hb, 1, HIDDEN) + Tp.reshape(1, N_BKT, HIDDEN))


def _idx_body(ts_ref, t_ref, mk_ref, idx_ref):
    f32 = jnp.float32
    ts = ts_ref[...]
    m = ts != 0
    hour = ts // 3600 % 24 + 1
    weekday = (ts // 86400 + 4) % 7 + 1
    hw = jnp.where(m, (hour - 1) * 7 + weekday, 0)
    bf = jnp.floor(jnp.log1p(t_ref[...].astype(f32))).astype(jnp.int32)
    bucket = jnp.clip(bf, 0, 21) + 1
    bucket = jnp.where(mk_ref[...], bucket, 0)
    idx_ref[...] = hw * N_BKT + bucket


def _tc_prologue(timestamps, t_from_prev, mask_i32,
                 h_emb, w_emb, i_emb, t_emb, norm_w, proj_w):
    B, L = timestamps.shape
    hb = N_HW // _GRID
    full = lambda s: pl.BlockSpec(s, lambda p: (0,) * len(s))
    table = pl.pallas_call(
        _table_body,
        grid=(_GRID,),
        in_specs=[
            full((25, 8)), full((8, 8)), full((3, 2)), full((130, 16)),
            full((1, 34)), full((HIDDEN, 34)),
        ],
        out_specs=pl.BlockSpec((hb, N_BKT, HIDDEN), lambda p: (p, 0, 0)),
        out_shape=jax.ShapeDtypeStruct((N_HW, N_BKT, HIDDEN), jnp.float32),
    )(h_emb, w_emb, i_emb, t_emb, norm_w.reshape(1, 34), proj_w)
    table = jnp.pad(table.reshape(N_ROWS, HIDDEN),
                    ((0, N_ROWS_PAD - N_ROWS), (0, 0)))
    bb = B // 8
    dspec = pl.BlockSpec((bb, L), lambda p: (p, 0))
    idx = pl.pallas_call(
        _idx_body,
        grid=(8,),
        in_specs=[dspec, dspec, dspec],
        out_specs=dspec,
        out_shape=jax.ShapeDtypeStruct((B, L), jnp.int32),
    )(timestamps, t_from_prev, mask_i32)
    return idx.reshape(B * L), table


_CH = 128
_NBUF = 4


def _make_gather(n_rows):
    info = plsc.get_sparse_core_info()
    nsub = info.num_subcores
    nworkers = info.num_cores * nsub
    per_w = n_rows // nworkers
    steps = per_w // _CH
    stage_rows = N_ROWS_PAD // nsub
    assert steps % _NBUF == 0 and steps >= 2 * _NBUF
    mesh = plsc.VectorSubcoreMesh(core_axis_name="c", subcore_axis_name="s")

    @functools.partial(
        pl.kernel,
        mesh=mesh,
        out_type=jax.ShapeDtypeStruct((n_rows, HIDDEN), jnp.float32),
        scratch_types=(
            [pltpu.VMEM((per_w,), jnp.int32),
             pltpu.VMEM((_NBUF, _CH, HIDDEN), jnp.float32),
             pltpu.VMEM_SHARED((N_ROWS_PAD, HIDDEN), jnp.float32)]
            + [pltpu.SemaphoreType.DMA] * (2 * _NBUF)
        ),
    )
    def gather(table_hbm, idx_hbm, out_hbm, idx_v, rows_v, table_sp, *sems):
        sg, sw = sems[:_NBUF], sems[_NBUF:]
        sid = lax.axis_index("s")
        wid = sid * info.num_cores + lax.axis_index("c")
        base = wid * per_w

        srow = pl.multiple_of(sid * stage_rows, stage_rows)
        pltpu.sync_copy(table_hbm.at[pl.ds(srow, stage_rows)],
                        table_sp.at[pl.ds(srow, stage_rows)])
        pltpu.sync_copy(idx_hbm.at[pl.ds(base, per_w)], idx_v)
        plsc.subcore_barrier()

        def start_gather(g, b, sem):
            off = pl.multiple_of(g * _CH, _CH)
            pltpu.async_copy(table_sp.at[idx_v.at[pl.ds(off, _CH)]],
                             rows_v.at[b], sem)

        def start_write(g, b, sem):
            off = pl.multiple_of(base + g * _CH, _CH)
            pltpu.async_copy(rows_v.at[b], out_hbm.at[pl.ds(off, _CH)], sem)

        def wait_chunk(sem):
            pltpu.make_async_copy(out_hbm.at[pl.ds(0, _CH)], rows_v.at[0], sem).wait()

        for b in range(_NBUF):
            start_gather(b, b, sg[b])
        wait_chunk(sg[0])
        start_write(0, 0, sw[0])

        def outer(og, carry):
            for b in range(_NBUF):
                g = og * _NBUF + b
                wait_chunk(sw[b])
                start_gather(g, b, sg[b])
                b2 = (b + 1) % _NBUF
                wait_chunk(sg[b2])
                start_write(g - (_NBUF - 1), b2, sw[b2])
            return carry

        lax.fori_loop(1, steps // _NBUF, outer, 0)

        for j in range(steps - (_NBUF - 1), steps):
            b = j % _NBUF
            wait_chunk(sg[b])
            start_write(j, b, sw[b])
        for b in range(_NBUF):
            wait_chunk(sw[b])

    return gather


def kernel(timestamps, t_from_prev, mask, hour_emb, weekday_emb, isweekend_emb,
           t_from_prev_emb, norm_w, proj_w):
    B, L = timestamps.shape
    idx, table = _tc_prologue(timestamps, t_from_prev, mask,
                              hour_emb, weekday_emb, isweekend_emb,
                              t_from_prev_emb, norm_w, proj_w)
    out = _make_gather(B * L)(table, idx)
    return out.reshape(B, L, HIDDEN)


# speedup vs baseline: 1.1700x; 1.0007x over previous
import functools

import jax
import jax.numpy as jnp
from jax import lax
from jax.experimental import pallas as pl
from jax.experimental.pallas import tpu as pltpu
from jax.experimental.pallas import tpu_sc as plsc

HIDDEN = 128
N_HW = 169
N_BKT = 23
N_ROWS = N_HW * N_BKT
N_ROWS_PAD = 3968
_GRID = 13


def _dot_nt(a, b):
    return lax.dot_general(a, b, (((1,), (1,)), ((), ())),
                           preferred_element_type=jnp.float32)


def _table_body(h_emb, w_emb, i_emb, t_emb, norm2d, proj_w, tab_ref):
    f32 = jnp.float32
    hb = tab_ref.shape[0]
    pid = pl.program_id(0)
    i = pid * hb + lax.broadcasted_iota(jnp.int32, (hb, 1), 0)
    valid = i >= 1
    hr = jnp.where(valid, (i - 1) // 7 + 1, 0)
    wd = jnp.where(valid, (i - 1) % 7 + 1, 0)
    iw = jnp.where(valid, jnp.where(wd >= 6, 2, 1), 0)

    oh_h = (hr == lax.broadcasted_iota(jnp.int32, (hb, 25), 1)).astype(f32)
    oh_w = (wd == lax.broadcasted_iota(jnp.int32, (hb, 8), 1)).astype(f32)
    oh_i = (iw == lax.broadcasted_iota(jnp.int32, (hb, 3), 1)).astype(f32)

    nrm = norm2d[...]
    prj = proj_w[...]
    t_sl = t_emb[0:N_BKT, :]
    Hp = _dot_nt(h_emb[...] * nrm[:, 8:16], prj[:, 8:16])
    Wp = _dot_nt(w_emb[...] * nrm[:, 0:8], prj[:, 0:8])
    Ip = _dot_nt(i_emb[...] * nrm[:, 16:18], prj[:, 16:18])
    Tp = _dot_nt(t_sl * nrm[:, 18:34], prj[:, 18:34])

    A = (jnp.dot(oh_w, Wp, preferred_element_type=f32)
         + jnp.dot(oh_h, Hp, preferred_element_type=f32)
         + jnp.dot(oh_i, Ip, preferred_element_type=f32))

    ss_h = jnp.sum(h_emb[...] * h_emb[...], axis=1, keepdims=True)
    ss_w = jnp.sum(w_emb[...] * w_emb[...], axis=1, keepdims=True)
    ss_i = jnp.sum(i_emb[...] * i_emb[...], axis=1, keepdims=True)
    ssA = (jnp.dot(oh_w, ss_w, preferred_element_type=f32)
           + jnp.dot(oh_h, ss_h, preferred_element_type=f32)
           + jnp.dot(oh_i, ss_i, preferred_element_type=f32))
    ssC = jnp.sum(t_sl * t_sl, axis=1, keepdims=True)

    eps = jnp.float32(jnp.finfo(jnp.float32).eps)
    mean_sq = (ssA.reshape(hb, 1, 1) + ssC.reshape(1, N_BKT, 1)) * jnp.float32(1.0 / 34.0)
    rms = lax.rsqrt(mean_sq + eps)
    tab_ref[...] = rms * (A.reshape(hb, 1, HIDDEN) + Tp.reshape(1, N_BKT, HIDDEN))


def _idx_body(ts_ref, t_ref, mk_ref, idx_ref):
    f32 = jnp.float32
    ts = ts_ref[...]
    m = ts != 0
    d1 = ts // 3600
    hour = d1 % 24 + 1
    weekday = (d1 // 24 + 4) % 7 + 1
    hw = jnp.where(m, (hour - 1) * 7 + weekday, 0)
    bf = jnp.floor(jnp.log1p(t_ref[...].astype(f32))).astype(jnp.int32)
    bucket = jnp.clip(bf, 0, 21) + 1
    bucket = jnp.where(mk_ref[...], bucket, 0)
    idx_ref[...] = hw * N_BKT + bucket


def _tc_prologue(timestamps, t_from_prev, mask_i32,
                 h_emb, w_emb, i_emb, t_emb, norm_w, proj_w):
    B, L = timestamps.shape
    hb = N_HW // _GRID
    full = lambda s: pl.BlockSpec(s, lambda p: (0,) * len(s))
    table = pl.pallas_call(
        _table_body,
        grid=(_GRID,),
        in_specs=[
            full((25, 8)), full((8, 8)), full((3, 2)), full((130, 16)),
            full((1, 34)), full((HIDDEN, 34)),
        ],
        out_specs=pl.BlockSpec((hb, N_BKT, HIDDEN), lambda p: (p, 0, 0)),
        out_shape=jax.ShapeDtypeStruct((N_HW, N_BKT, HIDDEN), jnp.float32),
    )(h_emb, w_emb, i_emb, t_emb, norm_w.reshape(1, 34), proj_w)
    table = jnp.pad(table.reshape(N_ROWS, HIDDEN),
                    ((0, N_ROWS_PAD - N_ROWS), (0, 0)))
    bb = B // 8
    dspec = pl.BlockSpec((bb, L), lambda p: (p, 0))
    idx = pl.pallas_call(
        _idx_body,
        grid=(8,),
        in_specs=[dspec, dspec, dspec],
        out_specs=dspec,
        out_shape=jax.ShapeDtypeStruct((B, L), jnp.int32),
    )(timestamps, t_from_prev, mask_i32)
    return idx.reshape(B * L), table


_CH = 128
_NBUF = 4


def _make_gather(n_rows):
    info = plsc.get_sparse_core_info()
    nsub = info.num_subcores
    nworkers = info.num_cores * nsub
    per_w = n_rows // nworkers
    steps = per_w // _CH
    stage_rows = N_ROWS_PAD // nsub
    assert steps % _NBUF == 0 and steps >= 2 * _NBUF
    mesh = plsc.VectorSubcoreMesh(core_axis_name="c", subcore_axis_name="s")

    @functools.partial(
        pl.kernel,
        mesh=mesh,
        out_type=jax.ShapeDtypeStruct((n_rows, HIDDEN), jnp.float32),
        scratch_types=(
            [pltpu.VMEM((per_w,), jnp.int32),
             pltpu.VMEM((_NBUF, _CH, HIDDEN), jnp.float32),
             pltpu.VMEM_SHARED((N_ROWS_PAD, HIDDEN), jnp.float32)]
            + [pltpu.SemaphoreType.DMA] * (2 * _NBUF)
        ),
    )
    def gather(table_hbm, idx_hbm, out_hbm, idx_v, rows_v, table_sp, *sems):
        sg, sw = sems[:_NBUF], sems[_NBUF:]
        sid = lax.axis_index("s")
        wid = sid * info.num_cores + lax.axis_index("c")
        base = wid * per_w

        srow = pl.multiple_of(sid * stage_rows, stage_rows)
        pltpu.sync_copy(table_hbm.at[pl.ds(srow, stage_rows)],
                        table_sp.at[pl.ds(srow, stage_rows)])
        pltpu.sync_copy(idx_hbm.at[pl.ds(base, per_w)], idx_v)
        plsc.subcore_barrier()

        def start_gather(g, b, sem):
            off = pl.multiple_of(g * _CH, _CH)
            pltpu.async_copy(table_sp.at[idx_v.at[pl.ds(off, _CH)]],
                             rows_v.at[b], sem)

        def start_write(g, b, sem):
            off = pl.multiple_of(base + g * _CH, _CH)
            pltpu.async_copy(rows_v.at[b], out_hbm.at[pl.ds(off, _CH)], sem)

        def wait_chunk(sem):
            pltpu.make_async_copy(out_hbm.at[pl.ds(0, _CH)], rows_v.at[0], sem).wait()

        for b in range(_NBUF):
            start_gather(b, b, sg[b])
        wait_chunk(sg[0])
        start_write(0, 0, sw[0])

        def outer(og, carry):
            for b in range(_NBUF):
                g = og * _NBUF + b
                wait_chunk(sw[b])
                start_gather(g, b, sg[b])
                b2 = (b + 1) % _NBUF
                wait_chunk(sg[b2])
                start_write(g - (_NBUF - 1), b2, sw[b2])
            return carry

        lax.fori_loop(1, steps // _NBUF, outer, 0)

        for j in range(steps - (_NBUF - 1), steps):
            b = j % _NBUF
            wait_chunk(sg[b])
            start_write(j, b, sw[b])
        for b in range(_NBUF):
            wait_chunk(sw[b])

    return gather


def kernel(timestamps, t_from_prev, mask, hour_emb, weekday_emb, isweekend_emb,
           t_from_prev_emb, norm_w, proj_w):
    B, L = timestamps.shape
    idx, table = _tc_prologue(timestamps, t_from_prev, mask,
                              hour_emb, weekday_emb, isweekend_emb,
                              t_from_prev_emb, norm_w, proj_w)
    out = _make_gather(B * L)(table, idx)
    return out.reshape(B, L, HIDDEN)
